# baseline (speedup 1.0000x reference)
import functools

import jax
import jax.numpy as jnp
from jax import lax
from jax.experimental import pallas as pl
from jax.experimental.pallas import tpu as pltpu
from jax.experimental.pallas import tpu_sc as plsc

_NOPS = 8
_NBATCH = 2

_B_SC = 16
_NW = 32
_SPLITS = _NW // _B_SC
_CHUNK = 32
_NRING = 4


def _sc_pool_body(x_hbm, o_hbm, *rest):
    bufs = rest[:_NRING]
    acc = rest[_NRING]
    sems = rest[_NRING + 1:]
    B, HW, C = x_hbm.shape
    rows = HW // _SPLITS
    nch = rows // _CHUNK
    lg = C // 16
    c = lax.axis_index("c")
    s = lax.axis_index("s")
    wid = c * 16 + s
    bidx = wid // _SPLITS
    half = wid % _SPLITS
    batch = (B - _B_SC) + bidx
    base = half * rows

    def start(ch, k):
        return pltpu.async_copy(
            x_hbm.at[batch, pl.ds(base + ch * _CHUNK, _CHUNK), :],
            bufs[k], sems[k],
        )

    @pl.loop(0, lg)
    def _(j):
        acc[pl.ds(j * 16, 16)] = jnp.zeros((16,), jnp.float32)

    for k in range(_NRING):
        start(k, k).start()

    def accumulate(buf):
        @pl.loop(0, _CHUNK)
        def _(r):
            for j in range(lg):
                sl = pl.ds(j * 16, 16)
                acc[sl] = acc[sl] + buf[r, sl]

    def phase(p, carry):
        ch0 = p * _NRING
        for k in range(_NRING):
            ch = ch0 + k
            start(ch, k).wait()
            accumulate(bufs[k])

            @pl.when(ch + _NRING < nch)
            def _():
                start(ch + _NRING, k).start()

        return carry

    lax.fori_loop(0, nch // _NRING, phase, 0)
    pltpu.sync_copy(acc, o_hbm.at[half, bidx, :])


def _sc_pool(xt):
    B, HW, C = xt.shape
    mesh = plsc.VectorSubcoreMesh(core_axis_name="c", subcore_axis_name="s")
    kern = pl.kernel(
        _sc_pool_body,
        out_type=jax.ShapeDtypeStruct((_SPLITS, _B_SC, C), jnp.float32),
        mesh=mesh,
        scratch_types=(
            [pltpu.VMEM((_CHUNK, C), jnp.float32) for _ in range(_NRING)]
            + [pltpu.VMEM((C,), jnp.float32)]
            + [pltpu.SemaphoreType.DMA for _ in range(_NRING)]
        ),
    )
    return kern(xt)


def _tc_pool_body(*refs):
    x_refs = refs[:_NOPS]
    o_ref = refs[_NOPS]
    i = pl.program_id(0)
    for bb in range(_NBATCH):
        parts = [jnp.sum(xq[bb], axis=0, keepdims=True) for xq in x_refs]
        row = parts[0]
        for p in parts[1:]:
            row = row + p
        o_ref[pl.ds(i * _NBATCH + bb, 1), :] = row


def _tc_pool(xt, b_tc):
    B, HW, C = xt.shape
    hsl = HW // _NOPS

    def _xspec(q):
        return pl.BlockSpec((_NBATCH, hsl, C), lambda i, q=q: (i, q, 0))

    return pl.pallas_call(
        _tc_pool_body,
        grid=(b_tc // _NBATCH,),
        in_specs=[_xspec(q) for q in range(_NOPS)],
        out_specs=pl.BlockSpec((b_tc, C), lambda i: (0, 0)),
        out_shape=jax.ShapeDtypeStruct((b_tc, C), jnp.float32),
    )(*([xt] * _NOPS))


def _finish_body(ptc_ref, psc_ref, w_ref, b_ref, o_ref, *, hw):
    psum = psc_ref[0]
    for k in range(1, _SPLITS):
        psum = psum + psc_ref[k]
    pooled = jnp.concatenate([ptc_ref[...], psum], axis=0)
    logits = jax.lax.dot_general(
        pooled, w_ref[...], (((1,), (1,)), ((), ())),
        preferred_element_type=jnp.float32,
    )
    o_ref[...] = jax.nn.sigmoid(logits * (1.0 / hw) + b_ref[...])


def kernel(x, Wc, b):
    B, C, H, W = x.shape
    hw = H * W
    xt = jnp.transpose(x, (0, 2, 3, 1)).reshape(B, hw, C)
    b2 = b.reshape(1, C)
    b_tc = B - _B_SC

    psc = _sc_pool(xt)
    ptc = _tc_pool(xt, b_tc)

    out = pl.pallas_call(
        functools.partial(_finish_body, hw=hw),
        in_specs=[
            pl.BlockSpec(memory_space=pltpu.VMEM),
            pl.BlockSpec(memory_space=pltpu.VMEM),
            pl.BlockSpec(memory_space=pltpu.VMEM),
            pl.BlockSpec(memory_space=pltpu.VMEM),
        ],
        out_specs=pl.BlockSpec(memory_space=pltpu.VMEM),
        out_shape=jax.ShapeDtypeStruct((B, C), jnp.float32),
    )(ptc, psc, Wc, b2)
    return out.reshape(B, 1, C, 1, 1)


# speedup vs baseline: 2.2803x; 2.2803x over previous
import jax
import jax.numpy as jnp
from jax.experimental import pallas as pl
from jax.experimental.pallas import tpu as pltpu

_NOPS = 16
_NBATCH = 2


def _gate_body(*refs):
    x_refs = refs[:_NOPS]
    w_ref, b_ref, o_ref, pooled_ref = refs[_NOPS:]
    i = pl.program_id(0)
    for bb in range(_NBATCH):
        parts = [jnp.sum(xq[bb], axis=0, keepdims=True) for xq in x_refs]
        row = parts[0]
        for p in parts[1:]:
            row = row + p
        pooled_ref[pl.ds(i * _NBATCH + bb, 1), :] = row

    @pl.when(i == pl.num_programs(0) - 1)
    def _():
        pooled = pooled_ref[...]
        logits = jax.lax.dot_general(
            pooled, w_ref[...], (((1,), (1,)), ((), ())),
            preferred_element_type=jnp.float32,
        )
        scale = 1.0 / (x_refs[0].shape[1] * _NOPS)
        o_ref[...] = jax.nn.sigmoid(logits * scale + b_ref[...])


def kernel(x, Wc, b):
    B, C, H, W = x.shape
    hw = H * W
    hsl = hw // _NOPS
    xt = jnp.transpose(x, (0, 2, 3, 1)).reshape(B, hw, C)
    b2 = b.reshape(1, C)

    def _xspec(q):
        return pl.BlockSpec((_NBATCH, hsl, C), lambda i, q=q: (i, q, 0))

    out = pl.pallas_call(
        _gate_body,
        grid=(B // _NBATCH,),
        in_specs=[_xspec(q) for q in range(_NOPS)] + [
            pl.BlockSpec((C, C), lambda i: (0, 0)),
            pl.BlockSpec((1, C), lambda i: (0, 0)),
        ],
        out_specs=pl.BlockSpec((B, C), lambda i: (0, 0)),
        out_shape=jax.ShapeDtypeStruct((B, C), jnp.float32),
        scratch_shapes=[pltpu.VMEM((B, C), jnp.float32)],
    )(*([xt] * _NOPS), Wc, b2)
    return out.reshape(B, 1, C, 1, 1)


# speedup vs baseline: 2.3829x; 1.0450x over previous
import jax
import jax.numpy as jnp
from jax.experimental import pallas as pl
from jax.experimental.pallas import tpu as pltpu

_NOPS = 8
_NBATCH = 4


def _gate_body(*refs):
    x_refs = refs[:_NOPS]
    w_ref, b_ref, o_ref, pooled_ref = refs[_NOPS:]
    i = pl.program_id(0)
    for bb in range(_NBATCH):
        parts = [jnp.sum(xq[bb], axis=0, keepdims=True) for xq in x_refs]
        row = parts[0]
        for p in parts[1:]:
            row = row + p
        pooled_ref[pl.ds(i * _NBATCH + bb, 1), :] = row

    @pl.when(i == pl.num_programs(0) - 1)
    def _():
        pooled = pooled_ref[...]
        logits = jax.lax.dot_general(
            pooled, w_ref[...], (((1,), (1,)), ((), ())),
            preferred_element_type=jnp.float32,
        )
        scale = 1.0 / (x_refs[0].shape[1] * _NOPS)
        o_ref[...] = jax.nn.sigmoid(logits * scale + b_ref[...])


def kernel(x, Wc, b):
    B, C, H, W = x.shape
    hw = H * W
    hsl = hw // _NOPS
    xt = jnp.transpose(x, (0, 2, 3, 1)).reshape(B, hw, C)
    b2 = b.reshape(1, C)

    def _xspec(q):
        return pl.BlockSpec((_NBATCH, hsl, C), lambda i, q=q: (i, q, 0))

    out = pl.pallas_call(
        _gate_body,
        grid=(B // _NBATCH,),
        in_specs=[_xspec(q) for q in range(_NOPS)] + [
            pl.BlockSpec((C, C), lambda i: (0, 0)),
            pl.BlockSpec((1, C), lambda i: (0, 0)),
        ],
        out_specs=pl.BlockSpec((B, C), lambda i: (0, 0)),
        out_shape=jax.ShapeDtypeStruct((B, C), jnp.float32),
        scratch_shapes=[pltpu.VMEM((B, C), jnp.float32)],
    )(*([xt] * _NOPS), Wc, b2)
    return out.reshape(B, 1, C, 1, 1)
